# baseline (speedup 1.0000x reference)
import functools

import jax
import jax.numpy as jnp
from jax import lax
from jax.experimental import pallas as pl
from jax.experimental.pallas import tpu as pltpu
from jax.experimental.pallas import tpu_sc as plsc

NUM_EMB = 1_000_000
DIM = 64
NC = 2
NS = 16
NW = NC * NS
TOT = 4096 * 200
BPW = TOT // NW
CHUNK = 128
NCH = BPW // CHUNK
NBUF = 8
NGRP = NCH // NBUF


def _emb_gather(table, idx):
    mesh = plsc.VectorSubcoreMesh(
        core_axis_name="c", subcore_axis_name="s", num_cores=NC, num_subcores=NS
    )

    @functools.partial(
        pl.kernel,
        out_type=jax.ShapeDtypeStruct((NW, NCH, CHUNK, DIM), jnp.float32),
        mesh=mesh,
        compiler_params=pltpu.CompilerParams(use_tc_tiling_on_sc=False),
        scratch_types=[
            pltpu.VMEM((NCH, CHUNK), jnp.int32),
            [pltpu.VMEM((CHUNK, DIM), jnp.float32) for _ in range(NBUF)],
            pltpu.SemaphoreType.DMA((NBUF,)),
            pltpu.SemaphoreType.DMA((NBUF,)),
        ],
    )
    def body(table_hbm, idx_hbm, out_hbm, idx_v, bufs, gsem, osem):
        wid = lax.axis_index("s") * NC + lax.axis_index("c")
        pltpu.sync_copy(idx_hbm.at[wid], idx_v)

        for b in range(NBUF):
            pltpu.async_copy(table_hbm.at[idx_v.at[b]], bufs[b], gsem.at[b])

        @pl.loop(0, NGRP - 1)
        def _(grp):
            for b in range(NBUF):
                ch = grp * NBUF + b
                pltpu.make_async_copy(
                    table_hbm.at[idx_v.at[ch]], bufs[b], gsem.at[b]
                ).wait()
                pltpu.async_copy(bufs[b], out_hbm.at[wid, ch], osem.at[b])
            for b in range(NBUF):
                ch = grp * NBUF + b
                pltpu.make_async_copy(
                    bufs[b], out_hbm.at[wid, ch], osem.at[b]
                ).wait()
                pltpu.async_copy(
                    table_hbm.at[idx_v.at[ch + NBUF]], bufs[b], gsem.at[b]
                )

        last = (NGRP - 1) * NBUF
        for b in range(NBUF):
            pltpu.make_async_copy(
                table_hbm.at[idx_v.at[last + b]], bufs[b], gsem.at[b]
            ).wait()
            pltpu.async_copy(bufs[b], out_hbm.at[wid, last + b], osem.at[b])
        for b in range(NBUF):
            pltpu.make_async_copy(
                bufs[b], out_hbm.at[wid, last + b], osem.at[b]
            ).wait()

    return body(table, idx)


def kernel(embeddings, token_ids):
    b, s = token_ids.shape
    idx = token_ids.reshape(NW, NCH, CHUNK).astype(jnp.int32)
    out = _emb_gather(embeddings, idx)
    return out.reshape(b, s, DIM)


# speedup vs baseline: 1.4882x; 1.4882x over previous
import functools

import jax
import jax.numpy as jnp
from jax import lax
from jax.experimental import pallas as pl
from jax.experimental.pallas import tpu as pltpu
from jax.experimental.pallas import tpu_sc as plsc

NUM_EMB = 1_000_000
DIM = 64
NC = 2
NS = 16
NW = NC * NS
TOT = 4096 * 200
BPW = TOT // NW
CHUNK = 128
NCH = BPW // CHUNK
NBUF = 4
NGRP = NCH // NBUF


def _emb_gather(table, idx):
    mesh = plsc.VectorSubcoreMesh(
        core_axis_name="c", subcore_axis_name="s", num_cores=NC, num_subcores=NS
    )

    @functools.partial(
        pl.kernel,
        out_type=jax.ShapeDtypeStruct((NW, NCH, CHUNK, DIM), jnp.float32),
        mesh=mesh,
        scratch_types=[
            pltpu.VMEM((NCH, CHUNK), jnp.int32),
            [pltpu.VMEM((CHUNK, DIM), jnp.float32) for _ in range(NBUF)],
            pltpu.SemaphoreType.DMA((NBUF,)),
            pltpu.SemaphoreType.DMA((NBUF,)),
        ],
    )
    def body(table_hbm, idx_hbm, out_hbm, idx_v, bufs, gsem, osem):
        wid = lax.axis_index("s") * NC + lax.axis_index("c")
        pltpu.sync_copy(idx_hbm.at[wid], idx_v)

        def fire(ch, b):
            for j16 in range(CHUNK // 16):
                v = idx_v[ch, pl.ds(j16 * 16, 16)]
                for j in range(16):
                    pltpu.async_copy(
                        table_hbm.at[v[j]], bufs[b].at[j16 * 16 + j], gsem.at[b]
                    )

        def wait_gather(b):
            pltpu.make_async_copy(
                table_hbm.at[pl.ds(0, CHUNK)], bufs[b], gsem.at[b]
            ).wait()

        for b in range(NBUF):
            fire(b, b)

        @pl.loop(0, NGRP - 1)
        def _(grp):
            for b in range(NBUF):
                ch = grp * NBUF + b
                wait_gather(b)
                pltpu.async_copy(bufs[b], out_hbm.at[wid, ch], osem.at[b])
            for b in range(NBUF):
                ch = grp * NBUF + b
                pltpu.make_async_copy(
                    bufs[b], out_hbm.at[wid, ch], osem.at[b]
                ).wait()
                fire(ch + NBUF, b)

        last = (NGRP - 1) * NBUF
        for b in range(NBUF):
            wait_gather(b)
            pltpu.async_copy(bufs[b], out_hbm.at[wid, last + b], osem.at[b])
        for b in range(NBUF):
            pltpu.make_async_copy(
                bufs[b], out_hbm.at[wid, last + b], osem.at[b]
            ).wait()

    return body(table, idx)


def kernel(embeddings, token_ids):
    b, s = token_ids.shape
    idx = token_ids.reshape(NW, NCH, CHUNK).astype(jnp.int32)
    out = _emb_gather(embeddings, idx)
    return out.reshape(b, s, DIM)
